# Initial kernel scaffold; baseline (speedup 1.0000x reference)
import jax
import jax.numpy as jnp
from jax.experimental import pallas as pl


def kernel(input, rois):
    raise NotImplementedError("write your pallas kernel here")


# speedup vs baseline: 10.0836x; 10.0836x over previous
import functools

import jax
import jax.numpy as jnp
from jax import lax
from jax.experimental import pallas as pl
from jax.experimental.pallas import tpu as pltpu
from jax.experimental.pallas import tpu_sc as plsc

OH, OW = 7, 7
SCALE = 0.125
WMAX = 40
NCC = 8
CCW = 32
NROI = 512
NC, NS = 2, 16
NW = NC * NS
RPW = NROI // NW
NT = RPW * NCC


def _sc_body(xin_hbm, boxes_hbm, out_hbm, win0, win1, obuf, boxes_v, sem0, sem1):
    cid = lax.axis_index("c")
    sid = lax.axis_index("s")
    wid = sid * NC + cid
    base = wid * RPW

    pltpu.sync_copy(boxes_hbm.at[pl.ds(base, RPW)], boxes_v)

    def issue(t, buf, sem):
        r = t // NCC
        cc = t % NCC
        v = boxes_v[r, :]
        b, wsy, wsx = v[0], v[5], v[6]
        return pltpu.async_copy(
            xin_hbm.at[cc, b, pl.ds(wsy, WMAX), pl.ds(wsx, WMAX), :], buf, sem
        )

    def wait(t, buf, sem):
        r = t // NCC
        cc = t % NCC
        v = boxes_v[r, :]
        b, wsy, wsx = v[0], v[5], v[6]
        pltpu.make_async_copy(
            xin_hbm.at[cc, b, pl.ds(wsy, WMAX), pl.ds(wsx, WMAX), :], buf, sem
        ).wait()

    def compute(t, win):
        r = t // NCC
        cc = t % NCC
        v = boxes_v[r, :]
        y1 = v[1]
        x1 = v[2]
        hr = v[3]
        wr = v[4]
        yo = y1 - v[5]
        xo = x1 - v[6]
        neg = jnp.full((16,), -jnp.inf, jnp.float32)

        def bin_i(i, _):
            hs = (i * hr) // OH
            he = ((i + 1) * hr + (OH - 1)) // OH

            def bin_j(j, _):
                ws = (j * wr) // OW
                we = ((j + 1) * wr + (OW - 1)) // OW

                def yl(y, accs):
                    def xl(x, accs):
                        a0, a1 = accs
                        v0 = win[y, x, pl.ds(0, 16)]
                        v1 = win[y, x, pl.ds(16, 16)]
                        return jnp.maximum(a0, v0), jnp.maximum(a1, v1)

                    return lax.fori_loop(xo + ws, xo + we, xl, accs)

                a0, a1 = lax.fori_loop(yo + hs, yo + he, yl, (neg, neg))
                bi = i * OW + j
                obuf[bi, pl.ds(0, 16)] = a0
                obuf[bi, pl.ds(16, 16)] = a1
                return 0

            lax.fori_loop(0, OW, bin_j, 0)
            return 0

        lax.fori_loop(0, OH, bin_i, 0)
        pltpu.sync_copy(obuf, out_hbm.at[base + r, cc])

    issue(0, win0, sem0)

    def pair(p, _):
        t0 = 2 * p
        t1 = t0 + 1
        issue(t1, win1, sem1)
        wait(t0, win0, sem0)
        compute(t0, win0)

        @pl.when(p < NT // 2 - 1)
        def _():
            issue(t0 + 2, win0, sem0)

        wait(t1, win1, sem1)
        compute(t1, win1)
        return 0

    lax.fori_loop(0, NT // 2, pair, 0)


@functools.partial(jax.jit, static_argnums=())
def _roi_pool_sc(xin, boxes):
    mesh = plsc.VectorSubcoreMesh(core_axis_name="c", subcore_axis_name="s")
    f = functools.partial(
        pl.kernel,
        out_type=jax.ShapeDtypeStruct((NROI, NCC, OH * OW, CCW), jnp.float32),
        mesh=mesh,
        scratch_types=[
            pltpu.VMEM((WMAX, WMAX, CCW), jnp.float32),
            pltpu.VMEM((WMAX, WMAX, CCW), jnp.float32),
            pltpu.VMEM((OH * OW, CCW), jnp.float32),
            pltpu.VMEM((RPW, 16), jnp.int32),
            pltpu.SemaphoreType.DMA,
            pltpu.SemaphoreType.DMA,
        ],
        compiler_params=pltpu.CompilerParams(use_tc_tiling_on_sc=False),
    )(_sc_body)
    return f(xin, boxes)


def kernel(input, rois):
    n, c, h, w = input.shape
    xin = input.reshape(n, NCC, CCW, h, w).transpose(1, 0, 3, 4, 2)

    b = rois[:, 0].astype(jnp.int32)
    x1 = (rois[:, 1] * SCALE).astype(jnp.int32)
    y1 = (rois[:, 2] * SCALE).astype(jnp.int32)
    x2 = (rois[:, 3] * SCALE).astype(jnp.int32)
    y2 = (rois[:, 4] * SCALE).astype(jnp.int32)
    hr = jnp.clip(y2 - y1, 1, WMAX)
    wr = jnp.clip(x2 - x1, 1, WMAX)
    b = jnp.clip(b, 0, n - 1)
    y1 = jnp.clip(y1, 0, h - 1)
    x1 = jnp.clip(x1, 0, w - 1)
    wsy = jnp.clip(jnp.minimum(y1, h - WMAX), 0, h - WMAX)
    wsx = jnp.clip(jnp.minimum(x1, w - WMAX), 0, w - WMAX)
    z = jnp.zeros_like(b)
    boxes = jnp.stack(
        [b, y1, x1, hr, wr, wsy, wsx, z, z, z, z, z, z, z, z, z], axis=1
    )

    out = _roi_pool_sc(xin, boxes)
    return (
        out.reshape(NROI, NCC, OH, OW, CCW)
        .transpose(0, 1, 4, 2, 3)
        .reshape(NROI, NCC * CCW, OH, OW)
    )


# speedup vs baseline: 10.8961x; 1.0806x over previous
import functools

import jax
import jax.numpy as jnp
from jax import lax
from jax.experimental import pallas as pl
from jax.experimental.pallas import tpu as pltpu
from jax.experimental.pallas import tpu_sc as plsc

OH, OW = 7, 7
SCALE = 0.125
WMAX = 40
NCC = 8
CCW = 32
NROI = 512
NC, NS = 2, 16
NW = NC * NS
RPW = NROI // NW
NT = RPW * NCC
NB = OH * OW


def _sc_body(xin_hbm, boxes_hbm, out_hbm, win0, win1, ob0, ob1, boxes_v,
             sem0, sem1, osem0, osem1):
    cid = lax.axis_index("c")
    sid = lax.axis_index("s")
    wid = sid * NC + cid
    base = wid * RPW

    pltpu.sync_copy(boxes_hbm.at[pl.ds(base, RPW)], boxes_v)

    def win_slice(t):
        r = t // NCC
        cc = t % NCC
        v = boxes_v[r, pl.ds(0, 16)]
        return xin_hbm.at[cc, v[0], pl.ds(v[1], WMAX), pl.ds(v[2], WMAX), :]

    def issue(t, buf, sem):
        return pltpu.async_copy(win_slice(t), buf, sem)

    def wait(t, buf, sem):
        pltpu.make_async_copy(win_slice(t), buf, sem).wait()

    def out_copy(t, ob, osem):
        r = t // NCC
        cc = t % NCC
        return pltpu.async_copy(ob, out_hbm.at[base + r, cc], osem)

    def out_wait(t, ob, osem):
        r = t // NCC
        cc = t % NCC
        pltpu.make_async_copy(ob, out_hbm.at[base + r, cc], osem).wait()

    def compute(r, cc, win, ob):
        v0 = boxes_v[r, pl.ds(0, 16)]
        v1 = boxes_v[r, pl.ds(16, 16)]
        v2 = boxes_v[r, pl.ds(32, 16)]
        v3 = boxes_v[r, pl.ds(48, 16)]
        neg = jnp.full((16,), -jnp.inf, jnp.float32)
        for i in range(OH):
            ys = v0[4 + i]
            ye = v1[i]
            for j in range(OW):
                xs = v2[j]
                xe = v3[j]

                def yl(y, accs):
                    def xl(x, accs):
                        a0, a1 = accs
                        w0 = win[y, x, pl.ds(0, 16)]
                        w1 = win[y, x, pl.ds(16, 16)]
                        return jnp.maximum(a0, w0), jnp.maximum(a1, w1)

                    return plsc.parallel_loop(xs, xe, carry=accs, unroll=2)(xl)

                a0, a1 = lax.fori_loop(ys, ye, yl, (neg, neg))
                bi = i * OW + j
                ob[bi, pl.ds(0, 16)] = a0
                ob[bi, pl.ds(16, 16)] = a1

    issue(0, win0, sem0)

    def roi_body(r, _):
        def pair(q, _):
            t0 = r * NCC + 2 * q
            cc0 = 2 * q
            issue(t0 + 1, win1, sem1)
            wait(t0, win0, sem0)

            @pl.when(t0 >= 2)
            def _():
                out_wait(t0 - 2, ob0, osem0)

            compute(r, cc0, win0, ob0)
            out_copy(t0, ob0, osem0)

            @pl.when(t0 + 2 < NT)
            def _():
                issue(t0 + 2, win0, sem0)

            wait(t0 + 1, win1, sem1)

            @pl.when(t0 >= 1)
            def _():
                out_wait(t0 - 1, ob1, osem1)

            compute(r, cc0 + 1, win1, ob1)
            out_copy(t0 + 1, ob1, osem1)
            return 0

        lax.fori_loop(0, NCC // 2, pair, 0)
        return 0

    lax.fori_loop(0, RPW, roi_body, 0)
    out_wait(NT - 2, ob0, osem0)
    out_wait(NT - 1, ob1, osem1)


@jax.jit
def _roi_pool_sc(xin, boxes):
    mesh = plsc.VectorSubcoreMesh(core_axis_name="c", subcore_axis_name="s")
    f = functools.partial(
        pl.kernel,
        out_type=jax.ShapeDtypeStruct((NROI, NCC, NB, CCW), jnp.float32),
        mesh=mesh,
        scratch_types=[
            pltpu.VMEM((WMAX, WMAX, CCW), jnp.float32),
            pltpu.VMEM((WMAX, WMAX, CCW), jnp.float32),
            pltpu.VMEM((NB, CCW), jnp.float32),
            pltpu.VMEM((NB, CCW), jnp.float32),
            pltpu.VMEM((RPW, 64), jnp.int32),
            pltpu.SemaphoreType.DMA,
            pltpu.SemaphoreType.DMA,
            pltpu.SemaphoreType.DMA,
            pltpu.SemaphoreType.DMA,
        ],
        compiler_params=pltpu.CompilerParams(use_tc_tiling_on_sc=False),
    )(_sc_body)
    return f(xin, boxes)


def kernel(input, rois):
    n, c, h, w = input.shape
    xin = input.reshape(n, NCC, CCW, h, w).transpose(1, 0, 3, 4, 2)

    b = jnp.clip(rois[:, 0].astype(jnp.int32), 0, n - 1)
    x1 = jnp.clip((rois[:, 1] * SCALE).astype(jnp.int32), 0, w - 1)
    y1 = jnp.clip((rois[:, 2] * SCALE).astype(jnp.int32), 0, h - 1)
    x2 = (rois[:, 3] * SCALE).astype(jnp.int32)
    y2 = (rois[:, 4] * SCALE).astype(jnp.int32)
    hr = jnp.clip(y2 - y1, 1, WMAX)
    wr = jnp.clip(x2 - x1, 1, WMAX)
    wsy = jnp.clip(jnp.minimum(y1, h - WMAX), 0, h - WMAX)
    wsx = jnp.clip(jnp.minimum(x1, w - WMAX), 0, w - WMAX)
    yo = y1 - wsy
    xo = x1 - wsx

    iarr = jnp.arange(OH, dtype=jnp.int32)
    hs = (iarr[None, :] * hr[:, None]) // OH
    he = ((iarr[None, :] + 1) * hr[:, None] + (OH - 1)) // OH
    ws = (iarr[None, :] * wr[:, None]) // OW
    we = ((iarr[None, :] + 1) * wr[:, None] + (OW - 1)) // OW
    ys = jnp.clip(yo[:, None] + hs, 0, WMAX)
    ye = jnp.clip(yo[:, None] + he, 0, WMAX)
    xs = jnp.clip(xo[:, None] + ws, 0, WMAX)
    xe = jnp.clip(xo[:, None] + we, 0, WMAX)

    z7 = jnp.zeros((NROI, 7), jnp.int32)
    z9 = jnp.zeros((NROI, 9), jnp.int32)
    boxes = jnp.concatenate(
        [
            b[:, None], wsy[:, None], wsx[:, None], z7[:, :1],
            ys, z7[:, :5],
            ye, z9,
            xs, z9,
            xe, z9,
        ],
        axis=1,
    )

    out = _roi_pool_sc(xin, boxes)
    return (
        out.reshape(NROI, NCC, OH, OW, CCW)
        .transpose(0, 1, 4, 2, 3)
        .reshape(NROI, NCC * CCW, OH, OW)
    )


# speedup vs baseline: 15.8341x; 1.4532x over previous
import functools

import jax
import jax.numpy as jnp
from jax import lax
from jax.experimental import pallas as pl
from jax.experimental.pallas import tpu as pltpu
from jax.experimental.pallas import tpu_sc as plsc

OH, OW = 7, 7
SCALE = 0.125
WMAX = 40
NCC = 8
CCW = 32
NROI = 512
NC, NS = 2, 16
NW = NC * NS
RPW = NROI // NW
NT = RPW * NCC
NB = OH * OW


def _sc_body(xin_hbm, boxes_hbm, out_hbm, win0, win1, ob0, ob1, strip, boxes_v,
             sem0, sem1, osem0, osem1):
    cid = lax.axis_index("c")
    sid = lax.axis_index("s")
    wid = sid * NC + cid
    base = wid * RPW

    pltpu.sync_copy(boxes_hbm.at[pl.ds(base, RPW)], boxes_v)

    def win_slice(t):
        r = t // NCC
        cc = t % NCC
        v = boxes_v[r, pl.ds(0, 16)]
        return xin_hbm.at[cc, v[0], pl.ds(v[1], WMAX), pl.ds(v[2], WMAX), :]

    def issue(t, buf, sem):
        return pltpu.async_copy(win_slice(t), buf, sem)

    def wait(t, buf, sem):
        pltpu.make_async_copy(win_slice(t), buf, sem).wait()

    def out_copy(t, ob, osem):
        r = t // NCC
        cc = t % NCC
        return pltpu.async_copy(ob, out_hbm.at[base + r, cc], osem)

    def out_wait(t, ob, osem):
        r = t // NCC
        cc = t % NCC
        pltpu.make_async_copy(ob, out_hbm.at[base + r, cc], osem).wait()

    def _maxtree(vals):
        while len(vals) > 1:
            nxt = [jnp.maximum(a, b) for a, b in zip(vals[::2], vals[1::2])]
            if len(vals) % 2:
                nxt.append(vals[-1])
            vals = nxt
        return vals[0]

    def compute(r, cc, win, strip, ob):
        v0 = boxes_v[r, pl.ds(0, 16)]
        v1 = boxes_v[r, pl.ds(16, 16)]
        v2 = boxes_v[r, pl.ds(32, 16)]
        v3 = boxes_v[r, pl.ds(48, 16)]
        xb = v2[0]
        xE = v3[6]
        for i in range(OH):
            ys = v0[4 + i]
            yl = v1[i] - 1
            yy = [jnp.minimum(ys + dy, yl) for dy in range(8)]

            def xloop(x, yy=yy, i=i):
                m0 = _maxtree([win[y, x, pl.ds(0, 16)] for y in yy])
                m1 = _maxtree([win[y, x, pl.ds(16, 16)] for y in yy])
                strip[i, x, pl.ds(0, 16)] = m0
                strip[i, x, pl.ds(16, 16)] = m1

            plsc.parallel_loop(xb, xE)(xloop)
        for j in range(OW):
            xs = v2[j]
            xl = v3[j] - 1
            xx = [jnp.minimum(xs + dx, xl) for dx in range(8)]
            for i in range(OH):
                m0 = _maxtree([strip[i, x, pl.ds(0, 16)] for x in xx])
                m1 = _maxtree([strip[i, x, pl.ds(16, 16)] for x in xx])
                bi = i * OW + j
                ob[bi, pl.ds(0, 16)] = m0
                ob[bi, pl.ds(16, 16)] = m1

    issue(0, win0, sem0)

    def roi_body(r, _):
        def pair(q, _):
            t0 = r * NCC + 2 * q
            cc0 = 2 * q
            issue(t0 + 1, win1, sem1)
            wait(t0, win0, sem0)

            @pl.when(t0 >= 2)
            def _():
                out_wait(t0 - 2, ob0, osem0)

            compute(r, cc0, win0, strip, ob0)
            out_copy(t0, ob0, osem0)

            @pl.when(t0 + 2 < NT)
            def _():
                issue(t0 + 2, win0, sem0)

            wait(t0 + 1, win1, sem1)

            @pl.when(t0 >= 1)
            def _():
                out_wait(t0 - 1, ob1, osem1)

            compute(r, cc0 + 1, win1, strip, ob1)
            out_copy(t0 + 1, ob1, osem1)
            return 0

        lax.fori_loop(0, NCC // 2, pair, 0)
        return 0

    lax.fori_loop(0, RPW, roi_body, 0)
    out_wait(NT - 2, ob0, osem0)
    out_wait(NT - 1, ob1, osem1)


@jax.jit
def _roi_pool_sc(xin, boxes):
    mesh = plsc.VectorSubcoreMesh(core_axis_name="c", subcore_axis_name="s")
    f = functools.partial(
        pl.kernel,
        out_type=jax.ShapeDtypeStruct((NROI, NCC, NB, CCW), jnp.float32),
        mesh=mesh,
        scratch_types=[
            pltpu.VMEM((WMAX, WMAX, CCW), jnp.float32),
            pltpu.VMEM((WMAX, WMAX, CCW), jnp.float32),
            pltpu.VMEM((NB, CCW), jnp.float32),
            pltpu.VMEM((NB, CCW), jnp.float32),
            pltpu.VMEM((OH, WMAX, CCW), jnp.float32),
            pltpu.VMEM((RPW, 64), jnp.int32),
            pltpu.SemaphoreType.DMA,
            pltpu.SemaphoreType.DMA,
            pltpu.SemaphoreType.DMA,
            pltpu.SemaphoreType.DMA,
        ],
        compiler_params=pltpu.CompilerParams(use_tc_tiling_on_sc=False),
    )(_sc_body)
    return f(xin, boxes)


def kernel(input, rois):
    n, c, h, w = input.shape
    xin = input.reshape(n, NCC, CCW, h, w).transpose(1, 0, 3, 4, 2)

    b = jnp.clip(rois[:, 0].astype(jnp.int32), 0, n - 1)
    x1 = jnp.clip((rois[:, 1] * SCALE).astype(jnp.int32), 0, w - 1)
    y1 = jnp.clip((rois[:, 2] * SCALE).astype(jnp.int32), 0, h - 1)
    x2 = (rois[:, 3] * SCALE).astype(jnp.int32)
    y2 = (rois[:, 4] * SCALE).astype(jnp.int32)
    hr = jnp.clip(y2 - y1, 1, WMAX)
    wr = jnp.clip(x2 - x1, 1, WMAX)
    wsy = jnp.clip(jnp.minimum(y1, h - WMAX), 0, h - WMAX)
    wsx = jnp.clip(jnp.minimum(x1, w - WMAX), 0, w - WMAX)
    yo = y1 - wsy
    xo = x1 - wsx

    iarr = jnp.arange(OH, dtype=jnp.int32)
    hs = (iarr[None, :] * hr[:, None]) // OH
    he = ((iarr[None, :] + 1) * hr[:, None] + (OH - 1)) // OH
    ws = (iarr[None, :] * wr[:, None]) // OW
    we = ((iarr[None, :] + 1) * wr[:, None] + (OW - 1)) // OW
    ys = jnp.clip(yo[:, None] + hs, 0, WMAX)
    ye = jnp.clip(yo[:, None] + he, 0, WMAX)
    xs = jnp.clip(xo[:, None] + ws, 0, WMAX)
    xe = jnp.clip(xo[:, None] + we, 0, WMAX)

    z7 = jnp.zeros((NROI, 7), jnp.int32)
    z9 = jnp.zeros((NROI, 9), jnp.int32)
    boxes = jnp.concatenate(
        [
            b[:, None], wsy[:, None], wsx[:, None], z7[:, :1],
            ys, z7[:, :5],
            ye, z9,
            xs, z9,
            xe, z9,
        ],
        axis=1,
    )

    out = _roi_pool_sc(xin, boxes)
    return (
        out.reshape(NROI, NCC, OH, OW, CCW)
        .transpose(0, 1, 4, 2, 3)
        .reshape(NROI, NCC * CCW, OH, OW)
    )
